# Initial kernel scaffold; baseline (speedup 1.0000x reference)
import jax
import jax.numpy as jnp
from jax.experimental import pallas as pl


def kernel(x, table):
    raise NotImplementedError("write your pallas kernel here")


# speedup vs baseline: 1.5009x; 1.5009x over previous
import functools

import jax
import jax.numpy as jnp
from jax import lax
from jax.experimental import pallas as pl
from jax.experimental.pallas import tpu as pltpu
from jax.experimental.pallas import tpu_sc as plsc


def _pos_enc(seq_len, d_model):
    pos = jnp.arange(seq_len, dtype=jnp.float32)[:, None]
    _2i = jnp.arange(0, d_model, 2, dtype=jnp.float32)
    enc = jnp.zeros((seq_len, d_model), dtype=jnp.float32)
    enc = enc.at[:, 0::2].set(jnp.sin(pos / 10000 ** (_2i / d_model)))
    enc = enc.at[:, 1::2].set(jnp.cos(pos / 10000 ** (_2i / d_model)))
    return enc


@functools.lru_cache(maxsize=None)
def _build(B, S, D):
    info = plsc.get_sparse_core_info()
    NC, NS, L = info.num_cores, info.num_subcores, info.num_lanes
    NW = NC * NS
    SPT = S // NW
    CS = 32
    NCHUNK = SPT // CS
    NV = D // L

    mesh = plsc.VectorSubcoreMesh(core_axis_name="c", subcore_axis_name="s")

    @functools.partial(
        pl.kernel,
        mesh=mesh,
        out_type=jax.ShapeDtypeStruct((B, S, D), jnp.float32),
        scratch_types=[
            pltpu.VMEM((B * SPT,), jnp.int32),
            pltpu.VMEM((CS, D), jnp.float32),
            pltpu.VMEM((CS, D), jnp.float32),
            pltpu.SemaphoreType.DMA,
        ],
    )
    def embed(x_hbm, table_hbm, enc_hbm, out_hbm, idx_v, ebuf, rbuf, sem):
        wid = lax.axis_index("s") * NC + lax.axis_index("c")
        s0 = wid * SPT
        for b in range(B):
            pltpu.sync_copy(x_hbm.at[b, pl.ds(s0, SPT)],
                            idx_v.at[pl.ds(b * SPT, SPT)])

        def chunk_body(c, carry):
            sc0 = s0 + c * CS
            pltpu.sync_copy(enc_hbm.at[pl.ds(sc0, CS)], ebuf)
            for b in range(B):
                idx_slice = idx_v.at[pl.ds(b * SPT + c * CS, CS)]
                pltpu.async_copy(table_hbm.at[idx_slice], rbuf, sem).wait()

                def row_body(i, c2):
                    for k in range(NV):
                        sl = pl.ds(k * L, L)
                        plsc.addupdate(rbuf.at[i, sl], ebuf[i, sl])
                    return c2

                lax.fori_loop(0, CS, row_body, 0)
                pltpu.sync_copy(rbuf, out_hbm.at[b, pl.ds(sc0, CS)])
            return carry

        lax.fori_loop(0, NCHUNK, chunk_body, 0)

    return embed


def kernel(x, table):
    B, S = x.shape
    _, D = table.shape
    enc = _pos_enc(S, D)
    return _build(B, S, D)(x.astype(jnp.int32), table, enc)
